# baseline (speedup 1.0000x reference)
import functools

import jax
import jax.numpy as jnp
from jax.experimental import pallas as pl
from jax.experimental.pallas import tpu as pltpu

N = 4096
NFEAT = 512
NHID = 256
NCLASS = 64

BK1 = 512
NB1 = N // BK1
BMD = 1024
NBD = N // BMD

ADJ_SCALE = 2048.0
F8 = jnp.float8_e4m3fn

_DN = (((1,), (1,)), ((), ()))


def _body(adj_ref, x_ref, w1_ref, b1_ref, w2_ref, b2_ref, out_ref,
          adj8_v, hacc_v, s2_v, z_v):
    k = pl.program_id(0)

    @pl.when(k < NB1)
    def _layer_phase():
        a32 = adj_ref[...]
        adj8_v[:, pl.ds(k * BK1, BK1)] = (a32 * ADJ_SCALE).astype(F8)
        xb = x_ref[...].astype(jnp.bfloat16)
        w1 = w1_ref[...].astype(jnp.bfloat16)
        s1_j = jnp.dot(xb, w1, preferred_element_type=jnp.float32
                       ).astype(jnp.bfloat16)
        part = jnp.dot(a32.astype(jnp.bfloat16), s1_j,
                       preferred_element_type=jnp.float32)

        @pl.when(k == 0)
        def _init():
            hacc_v[...] = part

        @pl.when(k > 0)
        def _acc():
            hacc_v[...] += part

    @pl.when(k >= NB1)
    def _decode_phase():
        d = k - NB1

        @pl.when(d == 0)
        def _finish_layer():
            h = jnp.maximum(hacc_v[...] + b1_ref[...], 0.0).astype(jnp.bfloat16)
            w2 = w2_ref[...].astype(jnp.bfloat16)
            s2_v[...] = jnp.dot(h, w2, preferred_element_type=jnp.float32
                                ).astype(jnp.bfloat16)

        @pl.when(d < NBD)
        def _z_diag():
            a = adj8_v[pl.ds(d * BMD, BMD), :].astype(jnp.bfloat16)
            acc = jnp.dot(a, s2_v[...], preferred_element_type=jnp.float32)
            zb = acc * (1.0 / ADJ_SCALE) + b2_ref[...]
            z_v[pl.ds(d * BMD, BMD), :] = zb.astype(jnp.bfloat16)

        i, j = _tile_ij(d)
        zi = z_v[pl.ds(i * BMD, BMD), :]
        zj = z_v[pl.ds(j * BMD, BMD), :]
        t = jax.lax.dot_general(zi, zj, _DN, preferred_element_type=jnp.float32)
        out_ref[...] = 0.5 * (1.0 + jnp.tanh(0.5 * t))


def _tile_ij(d):
    e = jnp.maximum(d - NBD, 0)
    i_off = e // (NBD - 1)
    jj = e % (NBD - 1)
    j_off = jj + (jj >= i_off).astype(jj.dtype)
    on_diag = d < NBD
    i = jnp.where(on_diag, d, i_off)
    j = jnp.where(on_diag, d, j_off)
    return i, j


def _out_map(k):
    d = jnp.maximum(k - NB1, 0)
    return _tile_ij(d)


@functools.partial(jax.jit)
def kernel(x, adj, W1, b1, W2, b2):
    b1r = b1.reshape(1, NHID)
    b2r = b2.reshape(1, NCLASS)

    a_pred = pl.pallas_call(
        _body,
        grid=(NB1 + NBD * NBD,),
        in_specs=[
            pl.BlockSpec((N, BK1), lambda k: (0, jnp.minimum(k, NB1 - 1))),
            pl.BlockSpec((BK1, NFEAT), lambda k: (jnp.minimum(k, NB1 - 1), 0)),
            pl.BlockSpec((NFEAT, NHID), lambda k: (0, 0)),
            pl.BlockSpec((1, NHID), lambda k: (0, 0)),
            pl.BlockSpec((NHID, NCLASS), lambda k: (0, 0)),
            pl.BlockSpec((1, NCLASS), lambda k: (0, 0)),
        ],
        out_specs=pl.BlockSpec((BMD, BMD), _out_map),
        out_shape=jax.ShapeDtypeStruct((N, N), jnp.float32),
        scratch_shapes=[
            pltpu.VMEM((N, N), F8),
            pltpu.VMEM((N, NHID), jnp.float32),
            pltpu.VMEM((N, NCLASS), jnp.bfloat16),
            pltpu.VMEM((N, NCLASS), jnp.bfloat16),
        ],
    )(adj, x, W1, b1r, W2, b2r)

    return a_pred


# speedup vs baseline: 1.1014x; 1.1014x over previous
import functools

import jax
import jax.numpy as jnp
from jax.experimental import pallas as pl
from jax.experimental.pallas import tpu as pltpu

N = 4096
NFEAT = 512
NHID = 256
NCLASS = 64

BM1 = 512
NB1 = N // BM1
BMD = 1024
NBD = N // BMD

ADJ_SCALE = 2048.0
S2_SCALE = 512.0
INV_Z = 1.0 / (ADJ_SCALE * S2_SCALE * 1.4142135623730951)
F8 = jnp.float8_e4m3fn

_DN = (((1,), (1,)), ((), ()))


def _body(adj_ref, x_ref, w1_ref, b1_ref, w2_ref, b2_ref, out_ref,
          adj8_v, s1_v, s2_v, z_v):
    k = pl.program_id(0)

    @pl.when(k == 0)
    def _compute_s1():
        xb = x_ref[...].astype(jnp.bfloat16)
        w1 = w1_ref[...].astype(jnp.bfloat16)
        s1_v[...] = jnp.dot(xb, w1, preferred_element_type=jnp.float32
                            ).astype(F8)

    @pl.when(k < NB1)
    def _layer_phase():
        a32 = adj_ref[...]
        a8 = (a32 * ADJ_SCALE).astype(F8)
        adj8_v[pl.ds(k * BM1, BM1), :] = a8
        acc = jnp.dot(a8, s1_v[...], preferred_element_type=jnp.float32)
        h = jnp.maximum(acc * (1.0 / ADJ_SCALE) + b1_ref[...], 0.0
                        ).astype(jnp.bfloat16)
        w2 = w2_ref[...].astype(jnp.bfloat16)
        s2 = jnp.dot(h, w2, preferred_element_type=jnp.float32)
        s2_v[pl.ds(k * BM1, BM1), :] = (s2 * S2_SCALE).astype(F8)

    @pl.when(k >= NB1)
    def _decode_phase():
        d = k - NB1

        @pl.when(d < NBD)
        def _z_diag():
            a8 = adj8_v[pl.ds(d * BMD, BMD), :]
            acc = jnp.dot(a8, s2_v[...], preferred_element_type=jnp.float32)
            zb = acc * INV_Z + b2_ref[...]
            z_v[pl.ds(d * BMD, BMD), :] = zb.astype(jnp.bfloat16)

        i, j = _tile_ij(d)
        zi = z_v[pl.ds(i * BMD, BMD), :]
        zj = z_v[pl.ds(j * BMD, BMD), :]
        t = jax.lax.dot_general(zi, zj, _DN, preferred_element_type=jnp.float32)
        out_ref[...] = 0.5 * (1.0 + jnp.tanh(t))


def _tile_ij(d):
    e = jnp.maximum(d - NBD, 0)
    i_off = e // (NBD - 1)
    jj = e % (NBD - 1)
    j_off = jj + (jj >= i_off).astype(jj.dtype)
    on_diag = d < NBD
    i = jnp.where(on_diag, d, i_off)
    j = jnp.where(on_diag, d, j_off)
    return i, j


def _out_map(k):
    d = jnp.maximum(k - NB1, 0)
    return _tile_ij(d)


@functools.partial(jax.jit)
def kernel(x, adj, W1, b1, W2, b2):
    b1r = b1.reshape(1, NHID)
    b2r = (b2 * (1.0 / 1.4142135623730951)).reshape(1, NCLASS)

    a_pred = pl.pallas_call(
        _body,
        grid=(NB1 + NBD * NBD,),
        in_specs=[
            pl.BlockSpec((BM1, N), lambda k: (jnp.minimum(k, NB1 - 1), 0)),
            pl.BlockSpec((N, NFEAT), lambda k: (0, 0)),
            pl.BlockSpec((NFEAT, NHID), lambda k: (0, 0)),
            pl.BlockSpec((1, NHID), lambda k: (0, 0)),
            pl.BlockSpec((NHID, NCLASS), lambda k: (0, 0)),
            pl.BlockSpec((1, NCLASS), lambda k: (0, 0)),
        ],
        out_specs=pl.BlockSpec((BMD, BMD), _out_map),
        out_shape=jax.ShapeDtypeStruct((N, N), jnp.float32),
        scratch_shapes=[
            pltpu.VMEM((N, N), F8),
            pltpu.VMEM((N, NHID), F8),
            pltpu.VMEM((N, NCLASS), F8),
            pltpu.VMEM((N, NCLASS), jnp.bfloat16),
        ],
    )(adj, x, W1, b1r, W2, b2r)

    return a_pred


# speedup vs baseline: 1.1126x; 1.0101x over previous
import functools

import jax
import jax.numpy as jnp
from jax.experimental import pallas as pl
from jax.experimental.pallas import tpu as pltpu

N = 4096
NFEAT = 512
NHID = 256
NCLASS = 64

BM1 = 512
NB1 = N // BM1
BMD = 1024
NBD = N // BMD

S2_SCALE = 512.0
INV_Z = 1.0 / (S2_SCALE * 1.4142135623730951)
F8 = jnp.float8_e5m2

_DN = (((1,), (1,)), ((), ()))


def _body(adj_ref, x_ref, w1_ref, b1_ref, w2_ref, b2_ref, out_ref,
          adj8_v, s1_v, s2_v, z_v):
    k = pl.program_id(0)

    @pl.when(k == 0)
    def _compute_s1():
        xb = x_ref[...].astype(jnp.bfloat16)
        w1 = w1_ref[...].astype(jnp.bfloat16)
        s1_v[...] = jnp.dot(xb, w1, preferred_element_type=jnp.float32
                            ).astype(F8)

    @pl.when(k < NB1)
    def _layer_phase():
        a8 = adj_ref[...].astype(F8)
        adj8_v[pl.ds(k * BM1, BM1), :] = a8
        acc = jnp.dot(a8, s1_v[...], preferred_element_type=jnp.float32)
        h = jnp.maximum(acc + b1_ref[...], 0.0).astype(jnp.bfloat16)
        w2 = w2_ref[...].astype(jnp.bfloat16)
        s2 = jnp.dot(h, w2, preferred_element_type=jnp.float32)
        s2_v[pl.ds(k * BM1, BM1), :] = (s2 * S2_SCALE).astype(F8)

    @pl.when(k >= NB1)
    def _decode_phase():
        d = k - NB1

        @pl.when(d < NBD)
        def _z_diag():
            a8 = adj8_v[pl.ds(d * BMD, BMD), :]
            acc = jnp.dot(a8, s2_v[...], preferred_element_type=jnp.float32)
            zb = acc * INV_Z + b2_ref[...]
            z_v[pl.ds(d * BMD, BMD), :] = zb.astype(jnp.bfloat16)

        i, j = _tile_ij(d)
        zi = z_v[pl.ds(i * BMD, BMD), :]
        zj = z_v[pl.ds(j * BMD, BMD), :]
        t = jax.lax.dot_general(zi, zj, _DN, preferred_element_type=jnp.float32)
        out_ref[...] = 0.5 * (1.0 + jnp.tanh(t))


def _tile_ij(d):
    e = jnp.maximum(d - NBD, 0)
    i_off = e // (NBD - 1)
    jj = e % (NBD - 1)
    j_off = jj + (jj >= i_off).astype(jj.dtype)
    on_diag = d < NBD
    i = jnp.where(on_diag, d, i_off)
    j = jnp.where(on_diag, d, j_off)
    return i, j


def _out_map(k):
    d = jnp.maximum(k - NB1, 0)
    return _tile_ij(d)


@functools.partial(jax.jit)
def kernel(x, adj, W1, b1, W2, b2):
    b1r = b1.reshape(1, NHID)
    b2r = (b2 * (1.0 / 1.4142135623730951)).reshape(1, NCLASS)

    a_pred = pl.pallas_call(
        _body,
        grid=(NB1 + NBD * NBD,),
        in_specs=[
            pl.BlockSpec((BM1, N), lambda k: (jnp.minimum(k, NB1 - 1), 0)),
            pl.BlockSpec((N, NFEAT), lambda k: (0, 0)),
            pl.BlockSpec((NFEAT, NHID), lambda k: (0, 0)),
            pl.BlockSpec((1, NHID), lambda k: (0, 0)),
            pl.BlockSpec((NHID, NCLASS), lambda k: (0, 0)),
            pl.BlockSpec((1, NCLASS), lambda k: (0, 0)),
        ],
        out_specs=pl.BlockSpec((BMD, BMD), _out_map),
        out_shape=jax.ShapeDtypeStruct((N, N), jnp.float32),
        scratch_shapes=[
            pltpu.VMEM((N, N), F8),
            pltpu.VMEM((N, NHID), F8),
            pltpu.VMEM((N, NCLASS), F8),
            pltpu.VMEM((N, NCLASS), jnp.bfloat16),
        ],
    )(adj, x, W1, b1r, W2, b2r)

    return a_pred
